# Initial kernel scaffold; baseline (speedup 1.0000x reference)
import jax
import jax.numpy as jnp
from jax.experimental import pallas as pl


def kernel(h, edge_index, edge_features, n, W1, b1, W2, b2):
    raise NotImplementedError("write your pallas kernel here")


# speedup vs baseline: 3.7845x; 3.7845x over previous
import functools

import jax
import jax.numpy as jnp
from jax import lax
from jax.experimental import pallas as pl
from jax.experimental.pallas import tpu as pltpu
from jax.experimental.pallas import tpu_sc as plsc

N_NODES = 10000
N_EDGES = 320000
D = 128
NC = 2
NS = 16
NW = NC * NS
C = 80
CPW = N_EDGES // NW // C
ZR = 80
RPT = 640


def _tc_node_transform(h, W1a, W1b):
    blk = 2000

    def body(h_ref, wa_ref, wb_ref, p_ref, q_ref):
        hb = h_ref[...]
        p_ref[...] = jnp.dot(hb, wa_ref[...], preferred_element_type=jnp.float32)
        q_ref[...] = jnp.dot(hb, wb_ref[...], preferred_element_type=jnp.float32)

    return pl.pallas_call(
        body,
        grid=(N_NODES // blk,),
        in_specs=[
            pl.BlockSpec((blk, D), lambda i: (i, 0)),
            pl.BlockSpec((D, D), lambda i: (0, 0)),
            pl.BlockSpec((D, D), lambda i: (0, 0)),
        ],
        out_specs=[pl.BlockSpec((blk, D), lambda i: (i, 0))] * 2,
        out_shape=[jax.ShapeDtypeStruct((N_NODES, D), jnp.float32)] * 2,
    )(h, W1a, W1b)


def _sc_gather_add(P, Q, rows3d, cols3d):
    mesh = plsc.VectorSubcoreMesh(core_axis_name="c", subcore_axis_name="s")

    @functools.partial(
        pl.kernel,
        out_type=jax.ShapeDtypeStruct((N_EDGES, D), jnp.float32),
        mesh=mesh,
        scratch_types=[
            pltpu.VMEM((CPW, C), jnp.int32),
            pltpu.VMEM((CPW, C), jnp.int32),
            pltpu.VMEM((C, D), jnp.float32),
            pltpu.VMEM((C, D), jnp.float32),
            pltpu.SemaphoreType.DMA,
            pltpu.SemaphoreType.DMA,
        ],
    )
    def k(p_hbm, q_hbm, r_hbm, c_hbm, t_hbm, idxa, idxb, bufa, bufb, sema, semb):
        wid = lax.axis_index("s") * NC + lax.axis_index("c")
        e0 = wid * CPW * C
        pltpu.sync_copy(r_hbm.at[wid], idxa)
        pltpu.sync_copy(c_hbm.at[wid], idxb)

        @pl.loop(0, CPW)
        def _chunk(j):
            ca = pltpu.async_copy(p_hbm.at[idxa.at[j]], bufa, sema)
            cb = pltpu.async_copy(q_hbm.at[idxb.at[j]], bufb, semb)
            ca.wait()
            cb.wait()

            @pl.loop(0, C)
            def _row(r):
                for v in range(D // 16):
                    sl = pl.ds(v * 16, 16)
                    bufa[r, sl] = bufa[r, sl] + bufb[r, sl]

            pltpu.sync_copy(bufa, t_hbm.at[pl.ds(e0 + j * C, C)])

    return k(P, Q, rows3d, cols3d)


def _tc_mlp(T, ef, W1e, b1, W2, b2):
    blk = 2560
    F = ef.shape[1]

    def body(t_ref, e_ref, we_ref, b1_ref, w2_ref, b2_ref, o_ref):
        pre = t_ref[...] + jnp.dot(e_ref[...], we_ref[...],
                                   preferred_element_type=jnp.float32)
        hid = jnp.maximum(pre + b1_ref[...], 0.0)
        o_ref[...] = jnp.dot(hid, w2_ref[...],
                             preferred_element_type=jnp.float32) + b2_ref[...]

    return pl.pallas_call(
        body,
        grid=(N_EDGES // blk,),
        in_specs=[
            pl.BlockSpec((blk, D), lambda i: (i, 0)),
            pl.BlockSpec((blk, F), lambda i: (i, 0)),
            pl.BlockSpec((F, D), lambda i: (0, 0)),
            pl.BlockSpec((1, D), lambda i: (0, 0)),
            pl.BlockSpec((D, D), lambda i: (0, 0)),
            pl.BlockSpec((1, D), lambda i: (0, 0)),
        ],
        out_specs=pl.BlockSpec((blk, D), lambda i: (i, 0)),
        out_shape=jax.ShapeDtypeStruct((N_EDGES, D), jnp.float32),
    )(T, ef, W1e, b1, W2, b2)


def _sc_scatter_add(M, rows3d):
    mesh = plsc.VectorSubcoreMesh(core_axis_name="c", subcore_axis_name="s")

    @functools.partial(
        pl.kernel,
        out_type=jax.ShapeDtypeStruct((NC * N_NODES, D), jnp.float32),
        mesh=mesh,
        scratch_types=[
            pltpu.VMEM((CPW, C), jnp.int32),
            pltpu.VMEM((C, D), jnp.float32),
            pltpu.VMEM((ZR, D), jnp.float32),
            pltpu.VMEM_SHARED((N_NODES, D), jnp.float32),
        ],
    )
    def k(m_hbm, r_hbm, out_hbm, idxv, msgv, zbuf, hacc):
        cid = lax.axis_index("c")
        sid = lax.axis_index("s")
        wid = sid * NC + cid

        @pl.loop(0, ZR)
        def _z(r):
            for v in range(D // 16):
                zbuf[r, pl.ds(v * 16, 16)] = jnp.zeros((16,), jnp.float32)

        r_base = sid * RPT
        n_cp = jnp.where(sid == NS - 1, (N_NODES - (NS - 1) * RPT) // ZR, RPT // ZR)

        @pl.loop(0, n_cp)
        def _zc(kk):
            pltpu.sync_copy(zbuf, hacc.at[pl.ds(r_base + kk * ZR, ZR)])

        plsc.subcore_barrier()

        pltpu.sync_copy(r_hbm.at[wid], idxv)
        e0 = wid * CPW * C

        @pl.loop(0, CPW)
        def _chunk(j):
            pltpu.sync_copy(m_hbm.at[pl.ds(e0 + j * C, C)], msgv)
            pltpu.sync_copy(msgv, hacc.at[idxv.at[j]], add=True)

        plsc.subcore_barrier()

        @pl.loop(0, n_cp)
        def _wb(kk):
            r0 = r_base + kk * ZR
            pltpu.sync_copy(hacc.at[pl.ds(r0, ZR)],
                            out_hbm.at[pl.ds(cid * N_NODES + r0, ZR)])

    return k(M, rows3d)


def _tc_combine(Hp, delta_row):
    blk = 2000

    def body(a_ref, b_ref, d_ref, o_ref):
        o_ref[...] = a_ref[...] + b_ref[...] + d_ref[...]

    nblk = N_NODES // blk
    return pl.pallas_call(
        body,
        grid=(nblk,),
        in_specs=[
            pl.BlockSpec((blk, D), lambda i: (i, 0)),
            pl.BlockSpec((blk, D), lambda i, n=nblk: (i + n, 0)),
            pl.BlockSpec((1, D), lambda i: (0, 0)),
        ],
        out_specs=pl.BlockSpec((blk, D), lambda i: (i, 0)),
        out_shape=jax.ShapeDtypeStruct((N_NODES, D), jnp.float32),
    )(Hp, Hp, delta_row)


def kernel(h, edge_index, edge_features, n, W1, b1, W2, b2):
    rows = edge_index[0].astype(jnp.int32)
    cols = edge_index[1].astype(jnp.int32)
    rows3d = rows.reshape(NW, CPW, C)
    cols3d = cols.reshape(NW, CPW, C)
    W1a = W1[:D]
    W1b = W1[D:2 * D]
    W1e = W1[2 * D:]
    b1r = b1.reshape(1, D)
    b2r = b2.reshape(1, D)

    P, Q = _tc_node_transform(h, W1a, W1b)
    T = _sc_gather_add(P, Q, rows3d, cols3d)
    M = _tc_mlp(T, edge_features, W1e, b1r, W2, b2r)
    Hp = _sc_scatter_add(M, rows3d)
    delta = (jnp.asarray(n) - N_NODES).astype(jnp.float32)
    delta_row = jnp.full((1, D), delta, dtype=jnp.float32)
    return _tc_combine(Hp, delta_row)


# speedup vs baseline: 5.0374x; 1.3311x over previous
import functools

import jax
import jax.numpy as jnp
from jax import lax
from jax.experimental import pallas as pl
from jax.experimental.pallas import tpu as pltpu
from jax.experimental.pallas import tpu_sc as plsc

N_NODES = 10000
N_EDGES = 320000
D = 128
NC = 2
NS = 16
NW = NC * NS
C = 80
CPW = N_EDGES // NW // C
ZR = 80
RPT = 640


def _tc_node_transform(h, W1a, W1b):
    blk = 2000

    def body(h_ref, wa_ref, wb_ref, p_ref, q_ref):
        hb = h_ref[...]
        p_ref[...] = jnp.dot(hb, wa_ref[...], preferred_element_type=jnp.float32)
        q_ref[...] = jnp.dot(hb, wb_ref[...], preferred_element_type=jnp.float32)

    return pl.pallas_call(
        body,
        grid=(N_NODES // blk,),
        in_specs=[
            pl.BlockSpec((blk, D), lambda i: (i, 0)),
            pl.BlockSpec((D, D), lambda i: (0, 0)),
            pl.BlockSpec((D, D), lambda i: (0, 0)),
        ],
        out_specs=[pl.BlockSpec((blk, D), lambda i: (i, 0))] * 2,
        out_shape=[jax.ShapeDtypeStruct((N_NODES, D), jnp.float32)] * 2,
    )(h, W1a, W1b)


def _sc_gather_add(P, Q, rows3d, cols3d):
    mesh = plsc.VectorSubcoreMesh(core_axis_name="c", subcore_axis_name="s")

    @functools.partial(
        pl.kernel,
        out_type=jax.ShapeDtypeStruct((N_EDGES, D), jnp.float32),
        mesh=mesh,
        scratch_types=[
            pltpu.VMEM((CPW, C), jnp.int32),
            pltpu.VMEM((CPW, C), jnp.int32),
            pltpu.VMEM((2, C, D), jnp.float32),
            pltpu.VMEM((2, C, D), jnp.float32),
            pltpu.SemaphoreType.DMA,
            pltpu.SemaphoreType.DMA,
            pltpu.SemaphoreType.DMA,
            pltpu.SemaphoreType.DMA,
            pltpu.SemaphoreType.DMA,
            pltpu.SemaphoreType.DMA,
        ],
    )
    def k(p_hbm, q_hbm, r_hbm, c_hbm, t_hbm, idxa, idxb, bufa, bufb,
          sema0, sema1, semb0, semb1, wsem0, wsem1):
        sema = (sema0, sema1)
        semb = (semb0, semb1)
        wsem = (wsem0, wsem1)
        wid = lax.axis_index("s") * NC + lax.axis_index("c")
        e0 = wid * CPW * C
        pltpu.sync_copy(r_hbm.at[wid], idxa)
        pltpu.sync_copy(c_hbm.at[wid], idxb)

        def issue(j, b):
            pltpu.async_copy(p_hbm.at[idxa.at[j]], bufa.at[b], sema[b])
            pltpu.async_copy(q_hbm.at[idxb.at[j]], bufb.at[b], semb[b])

        def wait_gather(j, b):
            pltpu.make_async_copy(p_hbm.at[idxa.at[j]], bufa.at[b], sema[b]).wait()
            pltpu.make_async_copy(q_hbm.at[idxb.at[j]], bufb.at[b], semb[b]).wait()

        def wait_write(j, b):
            pltpu.make_async_copy(bufa.at[b], t_hbm.at[pl.ds(e0 + j * C, C)],
                                  wsem[b]).wait()

        def add_and_write(j, b):
            @pl.loop(0, C)
            def _row(r):
                for v in range(D // 16):
                    sl = pl.ds(v * 16, 16)
                    bufa[b, r, sl] = bufa[b, r, sl] + bufb[b, r, sl]

            pltpu.async_copy(bufa.at[b], t_hbm.at[pl.ds(e0 + j * C, C)],
                             wsem[b])

        issue(0, 0)

        @pl.loop(0, CPW - 1, step=2)
        def _body(j0):
            for b in (0, 1):
                j = j0 + b
                if b == 0:
                    @pl.when(j0 >= 1)
                    def _w():
                        wait_write(j, 1 - b)
                else:
                    wait_write(j, 1 - b)
                issue(j + 1, 1 - b)
                wait_gather(j, b)
                add_and_write(j, b)

        j_last = CPW - 1
        wait_gather(j_last, 0)
        add_and_write(j_last, 0)
        wait_write(j_last, 0)
        wait_write(j_last, 1)

    return k(P, Q, rows3d, cols3d)


def _tc_mlp(T, ef, W1e, b1, W2, b2):
    blk = 2560
    F = ef.shape[1]

    def body(t_ref, e_ref, we_ref, b1_ref, w2_ref, b2_ref, o_ref):
        pre = t_ref[...] + jnp.dot(e_ref[...], we_ref[...],
                                   preferred_element_type=jnp.float32)
        hid = jnp.maximum(pre + b1_ref[...], 0.0)
        o_ref[...] = jnp.dot(hid, w2_ref[...],
                             preferred_element_type=jnp.float32) + b2_ref[...]

    return pl.pallas_call(
        body,
        grid=(N_EDGES // blk,),
        in_specs=[
            pl.BlockSpec((blk, D), lambda i: (i, 0)),
            pl.BlockSpec((blk, F), lambda i: (i, 0)),
            pl.BlockSpec((F, D), lambda i: (0, 0)),
            pl.BlockSpec((1, D), lambda i: (0, 0)),
            pl.BlockSpec((D, D), lambda i: (0, 0)),
            pl.BlockSpec((1, D), lambda i: (0, 0)),
        ],
        out_specs=pl.BlockSpec((blk, D), lambda i: (i, 0)),
        out_shape=jax.ShapeDtypeStruct((N_EDGES, D), jnp.float32),
    )(T, ef, W1e, b1, W2, b2)


def _sc_scatter_add(M, rows3d):
    mesh = plsc.VectorSubcoreMesh(core_axis_name="c", subcore_axis_name="s")

    @functools.partial(
        pl.kernel,
        out_type=jax.ShapeDtypeStruct((NC * N_NODES, D), jnp.float32),
        mesh=mesh,
        scratch_types=[
            pltpu.VMEM((CPW, C), jnp.int32),
            pltpu.VMEM((2, C, D), jnp.float32),
            pltpu.VMEM((ZR, D), jnp.float32),
            pltpu.VMEM_SHARED((N_NODES, D), jnp.float32),
            pltpu.SemaphoreType.DMA,
            pltpu.SemaphoreType.DMA,
        ],
    )
    def k(m_hbm, r_hbm, out_hbm, idxv, msgv, zbuf, hacc, rsem0, rsem1):
        cid = lax.axis_index("c")
        sid = lax.axis_index("s")
        wid = sid * NC + cid

        @pl.loop(0, ZR)
        def _z(r):
            for v in range(D // 16):
                zbuf[r, pl.ds(v * 16, 16)] = jnp.zeros((16,), jnp.float32)

        r_base = sid * RPT
        n_cp = jnp.where(sid == NS - 1, (N_NODES - (NS - 1) * RPT) // ZR, RPT // ZR)

        @pl.loop(0, n_cp)
        def _zc(kk):
            pltpu.sync_copy(zbuf, hacc.at[pl.ds(r_base + kk * ZR, ZR)])

        plsc.subcore_barrier()

        pltpu.sync_copy(r_hbm.at[wid], idxv)
        e0 = wid * CPW * C
        rsem = (rsem0, rsem1)

        def issue_read(j, b):
            pltpu.async_copy(m_hbm.at[pl.ds(e0 + j * C, C)], msgv.at[b], rsem[b])

        def wait_read(j, b):
            pltpu.make_async_copy(m_hbm.at[pl.ds(e0 + j * C, C)], msgv.at[b],
                                  rsem[b]).wait()

        issue_read(0, 0)

        @pl.loop(0, CPW - 1, step=2)
        def _chunk(j0):
            for b in (0, 1):
                j = j0 + b
                issue_read(j + 1, 1 - b)
                wait_read(j, b)
                pltpu.sync_copy(msgv.at[b], hacc.at[idxv.at[j]], add=True)

        j_last = CPW - 1
        wait_read(j_last, 0)
        pltpu.sync_copy(msgv.at[0], hacc.at[idxv.at[j_last]], add=True)

        plsc.subcore_barrier()

        @pl.loop(0, n_cp)
        def _wb(kk):
            r0 = r_base + kk * ZR
            pltpu.sync_copy(hacc.at[pl.ds(r0, ZR)],
                            out_hbm.at[pl.ds(cid * N_NODES + r0, ZR)])

    return k(M, rows3d)


def _tc_combine(Hp, delta_row):
    blk = 2000

    def body(a_ref, b_ref, d_ref, o_ref):
        o_ref[...] = a_ref[...] + b_ref[...] + d_ref[...]

    nblk = N_NODES // blk
    return pl.pallas_call(
        body,
        grid=(nblk,),
        in_specs=[
            pl.BlockSpec((blk, D), lambda i: (i, 0)),
            pl.BlockSpec((blk, D), lambda i, n=nblk: (i + n, 0)),
            pl.BlockSpec((1, D), lambda i: (0, 0)),
        ],
        out_specs=pl.BlockSpec((blk, D), lambda i: (i, 0)),
        out_shape=jax.ShapeDtypeStruct((N_NODES, D), jnp.float32),
    )(Hp, Hp, delta_row)


def kernel(h, edge_index, edge_features, n, W1, b1, W2, b2):
    rows = edge_index[0].astype(jnp.int32)
    cols = edge_index[1].astype(jnp.int32)
    rows3d = rows.reshape(NW, CPW, C)
    cols3d = cols.reshape(NW, CPW, C)
    W1a = W1[:D]
    W1b = W1[D:2 * D]
    W1e = W1[2 * D:]
    b1r = b1.reshape(1, D)
    b2r = b2.reshape(1, D)

    P, Q = _tc_node_transform(h, W1a, W1b)
    T = _sc_gather_add(P, Q, rows3d, cols3d)
    M = _tc_mlp(T, edge_features, W1e, b1r, W2, b2r)
    Hp = _sc_scatter_add(M, rows3d)
    delta = (jnp.asarray(n) - N_NODES).astype(jnp.float32)
    delta_row = jnp.full((1, D), delta, dtype=jnp.float32)
    return _tc_combine(Hp, delta_row)


# speedup vs baseline: 5.0672x; 1.0059x over previous
import functools

import jax
import jax.numpy as jnp
from jax import lax
from jax.experimental import pallas as pl
from jax.experimental.pallas import tpu as pltpu
from jax.experimental.pallas import tpu_sc as plsc

N_NODES = 10000
N_EDGES = 320000
D = 128
NC = 2
NS = 16
NW = NC * NS
C = 80
CPW = N_EDGES // NW // C
ZR = 80
RPT = 640


def _tc_node_transform(h, W1a, W1b):
    blk = 2000

    def body(h_ref, wa_ref, wb_ref, p_ref, q_ref):
        hb = h_ref[...]
        p_ref[...] = jnp.dot(hb, wa_ref[...], preferred_element_type=jnp.float32)
        q_ref[...] = jnp.dot(hb, wb_ref[...], preferred_element_type=jnp.float32)

    return pl.pallas_call(
        body,
        grid=(N_NODES // blk,),
        in_specs=[
            pl.BlockSpec((blk, D), lambda i: (i, 0)),
            pl.BlockSpec((D, D), lambda i: (0, 0)),
            pl.BlockSpec((D, D), lambda i: (0, 0)),
        ],
        out_specs=[pl.BlockSpec((blk, D), lambda i: (i, 0))] * 2,
        out_shape=[jax.ShapeDtypeStruct((N_NODES, D), jnp.float32)] * 2,
    )(h, W1a, W1b)


def _sc_gather_add(P, Q, rows3d, cols3d):
    mesh = plsc.VectorSubcoreMesh(core_axis_name="c", subcore_axis_name="s")

    @functools.partial(
        pl.kernel,
        out_type=jax.ShapeDtypeStruct((N_EDGES, D), jnp.float32),
        mesh=mesh,
        scratch_types=[
            pltpu.VMEM((CPW, C), jnp.int32),
            pltpu.VMEM((CPW, C), jnp.int32),
            pltpu.VMEM((2, C, D), jnp.float32),
            pltpu.VMEM((2, C, D), jnp.float32),
            pltpu.VMEM((2, C, D), jnp.float32),
            pltpu.SemaphoreType.DMA,
            pltpu.SemaphoreType.DMA,
            pltpu.SemaphoreType.DMA,
            pltpu.SemaphoreType.DMA,
            pltpu.SemaphoreType.DMA,
            pltpu.SemaphoreType.DMA,
        ],
    )
    def k(p_hbm, q_hbm, r_hbm, c_hbm, t_hbm, idxa, idxb, bufa, bufb, obuf,
          sema0, sema1, semb0, semb1, wsem0, wsem1):
        sema = (sema0, sema1)
        semb = (semb0, semb1)
        wsem = (wsem0, wsem1)
        wid = lax.axis_index("s") * NC + lax.axis_index("c")
        e0 = wid * CPW * C
        pltpu.sync_copy(r_hbm.at[wid], idxa)
        pltpu.sync_copy(c_hbm.at[wid], idxb)

        def issue(j, b):
            pltpu.async_copy(p_hbm.at[idxa.at[j]], bufa.at[b], sema[b])
            pltpu.async_copy(q_hbm.at[idxb.at[j]], bufb.at[b], semb[b])

        def wait_gather(j, b):
            pltpu.make_async_copy(p_hbm.at[idxa.at[j]], bufa.at[b], sema[b]).wait()
            pltpu.make_async_copy(q_hbm.at[idxb.at[j]], bufb.at[b], semb[b]).wait()

        def wait_write(j, b):
            pltpu.make_async_copy(obuf.at[b], t_hbm.at[pl.ds(e0 + j * C, C)],
                                  wsem[b]).wait()

        def step(j, b, issue_next):
            wait_gather(j, b)

            @pl.loop(0, C)
            def _row(r):
                for v in range(D // 16):
                    sl = pl.ds(v * 16, 16)
                    obuf[b, r, sl] = bufa[b, r, sl] + bufb[b, r, sl]

            if issue_next:
                issue(j + 2, b)
            pltpu.async_copy(obuf.at[b], t_hbm.at[pl.ds(e0 + j * C, C)],
                             wsem[b])

        issue(0, 0)
        issue(1, 1)

        @pl.loop(0, CPW - 3, step=2)
        def _body(j0):
            for b in (0, 1):
                j = j0 + b
                @pl.when(j0 >= 2)
                def _w():
                    wait_write(j - 2, b)
                step(j, b, True)

        j_t = CPW - 3
        wait_write(j_t - 2, 0)
        step(j_t, 0, True)
        wait_write(j_t - 1, 1)
        step(j_t + 1, 1, False)
        wait_write(j_t, 0)
        step(j_t + 2, 0, False)
        wait_write(j_t + 1, 1)
        wait_write(j_t + 2, 0)

    return k(P, Q, rows3d, cols3d)


def _tc_mlp(T, ef, W1e, b1, W2, b2):
    blk = 2560
    F = ef.shape[1]

    def body(t_ref, e_ref, we_ref, b1_ref, w2_ref, b2_ref, o_ref):
        pre = t_ref[...] + jnp.dot(e_ref[...], we_ref[...],
                                   preferred_element_type=jnp.float32)
        hid = jnp.maximum(pre + b1_ref[...], 0.0)
        o_ref[...] = jnp.dot(hid, w2_ref[...],
                             preferred_element_type=jnp.float32) + b2_ref[...]

    return pl.pallas_call(
        body,
        grid=(N_EDGES // blk,),
        in_specs=[
            pl.BlockSpec((blk, D), lambda i: (i, 0)),
            pl.BlockSpec((blk, F), lambda i: (i, 0)),
            pl.BlockSpec((F, D), lambda i: (0, 0)),
            pl.BlockSpec((1, D), lambda i: (0, 0)),
            pl.BlockSpec((D, D), lambda i: (0, 0)),
            pl.BlockSpec((1, D), lambda i: (0, 0)),
        ],
        out_specs=pl.BlockSpec((blk, D), lambda i: (i, 0)),
        out_shape=jax.ShapeDtypeStruct((N_EDGES, D), jnp.float32),
    )(T, ef, W1e, b1, W2, b2)


def _sc_scatter_add(M, rows3d):
    mesh = plsc.VectorSubcoreMesh(core_axis_name="c", subcore_axis_name="s")

    @functools.partial(
        pl.kernel,
        out_type=jax.ShapeDtypeStruct((NC * N_NODES, D), jnp.float32),
        mesh=mesh,
        scratch_types=[
            pltpu.VMEM((CPW, C), jnp.int32),
            pltpu.VMEM((2, C, D), jnp.float32),
            pltpu.VMEM((ZR, D), jnp.float32),
            pltpu.VMEM_SHARED((N_NODES, D), jnp.float32),
            pltpu.SemaphoreType.DMA,
            pltpu.SemaphoreType.DMA,
        ],
    )
    def k(m_hbm, r_hbm, out_hbm, idxv, msgv, zbuf, hacc, rsem0, rsem1):
        cid = lax.axis_index("c")
        sid = lax.axis_index("s")
        wid = sid * NC + cid

        @pl.loop(0, ZR)
        def _z(r):
            for v in range(D // 16):
                zbuf[r, pl.ds(v * 16, 16)] = jnp.zeros((16,), jnp.float32)

        r_base = sid * RPT
        n_cp = jnp.where(sid == NS - 1, (N_NODES - (NS - 1) * RPT) // ZR, RPT // ZR)

        @pl.loop(0, n_cp)
        def _zc(kk):
            pltpu.sync_copy(zbuf, hacc.at[pl.ds(r_base + kk * ZR, ZR)])

        plsc.subcore_barrier()

        pltpu.sync_copy(r_hbm.at[wid], idxv)
        e0 = wid * CPW * C
        rsem = (rsem0, rsem1)

        def issue_read(j, b):
            pltpu.async_copy(m_hbm.at[pl.ds(e0 + j * C, C)], msgv.at[b], rsem[b])

        def wait_read(j, b):
            pltpu.make_async_copy(m_hbm.at[pl.ds(e0 + j * C, C)], msgv.at[b],
                                  rsem[b]).wait()

        issue_read(0, 0)

        @pl.loop(0, CPW - 1, step=2)
        def _chunk(j0):
            for b in (0, 1):
                j = j0 + b
                issue_read(j + 1, 1 - b)
                wait_read(j, b)
                pltpu.sync_copy(msgv.at[b], hacc.at[idxv.at[j]], add=True)

        j_last = CPW - 1
        wait_read(j_last, 0)
        pltpu.sync_copy(msgv.at[0], hacc.at[idxv.at[j_last]], add=True)

        plsc.subcore_barrier()

        @pl.loop(0, n_cp)
        def _wb(kk):
            r0 = r_base + kk * ZR
            pltpu.sync_copy(hacc.at[pl.ds(r0, ZR)],
                            out_hbm.at[pl.ds(cid * N_NODES + r0, ZR)])

    return k(M, rows3d)


def _tc_combine(Hp, delta_row):
    blk = 2000

    def body(a_ref, b_ref, d_ref, o_ref):
        o_ref[...] = a_ref[...] + b_ref[...] + d_ref[...]

    nblk = N_NODES // blk
    return pl.pallas_call(
        body,
        grid=(nblk,),
        in_specs=[
            pl.BlockSpec((blk, D), lambda i: (i, 0)),
            pl.BlockSpec((blk, D), lambda i, n=nblk: (i + n, 0)),
            pl.BlockSpec((1, D), lambda i: (0, 0)),
        ],
        out_specs=pl.BlockSpec((blk, D), lambda i: (i, 0)),
        out_shape=jax.ShapeDtypeStruct((N_NODES, D), jnp.float32),
    )(Hp, Hp, delta_row)


def kernel(h, edge_index, edge_features, n, W1, b1, W2, b2):
    rows = edge_index[0].astype(jnp.int32)
    cols = edge_index[1].astype(jnp.int32)
    rows3d = rows.reshape(NW, CPW, C)
    cols3d = cols.reshape(NW, CPW, C)
    W1a = W1[:D]
    W1b = W1[D:2 * D]
    W1e = W1[2 * D:]
    b1r = b1.reshape(1, D)
    b2r = b2.reshape(1, D)

    P, Q = _tc_node_transform(h, W1a, W1b)
    T = _sc_gather_add(P, Q, rows3d, cols3d)
    M = _tc_mlp(T, edge_features, W1e, b1r, W2, b2r)
    Hp = _sc_scatter_add(M, rows3d)
    delta = (jnp.asarray(n) - N_NODES).astype(jnp.float32)
    delta_row = jnp.full((1, D), delta, dtype=jnp.float32)
    return _tc_combine(Hp, delta_row)


# speedup vs baseline: 5.0686x; 1.0003x over previous
import functools

import jax
import jax.numpy as jnp
from jax import lax
from jax.experimental import pallas as pl
from jax.experimental.pallas import tpu as pltpu
from jax.experimental.pallas import tpu_sc as plsc

N_NODES = 10000
N_EDGES = 320000
D = 128
NC = 2
NS = 16
NW = NC * NS
C = 80
CPW = N_EDGES // NW // C
ZR = 80
RPT = 640


def _tc_node_transform(h, W1a, W1b):
    blk = 2000

    def body(h_ref, wa_ref, wb_ref, p_ref, q_ref):
        hb = h_ref[...]
        p_ref[...] = jnp.dot(hb, wa_ref[...], preferred_element_type=jnp.float32)
        q_ref[...] = jnp.dot(hb, wb_ref[...], preferred_element_type=jnp.float32)

    return pl.pallas_call(
        body,
        grid=(N_NODES // blk,),
        in_specs=[
            pl.BlockSpec((blk, D), lambda i: (i, 0)),
            pl.BlockSpec((D, D), lambda i: (0, 0)),
            pl.BlockSpec((D, D), lambda i: (0, 0)),
        ],
        out_specs=[pl.BlockSpec((blk, D), lambda i: (i, 0))] * 2,
        out_shape=[jax.ShapeDtypeStruct((N_NODES, D), jnp.float32)] * 2,
    )(h, W1a, W1b)


def _sc_gather_add(P, Q, rows3, cols3):
    mesh = plsc.VectorSubcoreMesh(core_axis_name="c", subcore_axis_name="s")
    EW = N_EDGES // NW
    GC = 128
    NF = EW // GC
    TAIL = EW - NF * GC

    @functools.partial(
        pl.kernel,
        out_type=jax.ShapeDtypeStruct((N_EDGES, D), jnp.float32),
        mesh=mesh,
        scratch_types=[
            pltpu.VMEM((1, EW), jnp.int32),
            pltpu.VMEM((1, EW), jnp.int32),
            pltpu.VMEM((2, GC, D), jnp.float32),
            pltpu.VMEM((2, GC, D), jnp.float32),
            pltpu.VMEM((2, GC, D), jnp.float32),
            pltpu.SemaphoreType.DMA,
            pltpu.SemaphoreType.DMA,
            pltpu.SemaphoreType.DMA,
            pltpu.SemaphoreType.DMA,
            pltpu.SemaphoreType.DMA,
            pltpu.SemaphoreType.DMA,
        ],
    )
    def k(p_hbm, q_hbm, r_hbm, c_hbm, t_hbm, idxa, idxb, bufa, bufb, obuf,
          sema0, sema1, semb0, semb1, wsem0, wsem1):
        sema = (sema0, sema1)
        semb = (semb0, semb1)
        wsem = (wsem0, wsem1)
        wid = lax.axis_index("s") * NC + lax.axis_index("c")
        e0 = wid * EW
        pltpu.sync_copy(r_hbm.at[wid], idxa)
        pltpu.sync_copy(c_hbm.at[wid], idxb)

        def issue(j, b, n=GC):
            ia = idxa.at[0, pl.ds(j * GC, n)]
            ib = idxb.at[0, pl.ds(j * GC, n)]
            pltpu.async_copy(p_hbm.at[ia], bufa.at[b, pl.ds(0, n)], sema[b])
            pltpu.async_copy(q_hbm.at[ib], bufb.at[b, pl.ds(0, n)], semb[b])

        def wait_gather(j, b, n=GC):
            ia = idxa.at[0, pl.ds(j * GC, n)]
            ib = idxb.at[0, pl.ds(j * GC, n)]
            pltpu.make_async_copy(p_hbm.at[ia], bufa.at[b, pl.ds(0, n)],
                                  sema[b]).wait()
            pltpu.make_async_copy(q_hbm.at[ib], bufb.at[b, pl.ds(0, n)],
                                  semb[b]).wait()

        def wait_write(j, b, n=GC):
            pltpu.make_async_copy(obuf.at[b, pl.ds(0, n)],
                                  t_hbm.at[pl.ds(e0 + j * GC, n)],
                                  wsem[b]).wait()

        def step(j, b, issue_next, n=GC):
            wait_gather(j, b, n)

            @pl.loop(0, n)
            def _row(r):
                for v in range(D // 16):
                    sl = pl.ds(v * 16, 16)
                    obuf[b, r, sl] = bufa[b, r, sl] + bufb[b, r, sl]

            if issue_next:
                issue(*issue_next)
            pltpu.async_copy(obuf.at[b, pl.ds(0, n)],
                             t_hbm.at[pl.ds(e0 + j * GC, n)], wsem[b])

        issue(0, 0)
        issue(1, 1)

        @pl.loop(0, NF - 4, step=2)
        def _body(j0):
            for b in (0, 1):
                j = j0 + b
                @pl.when(j0 >= 2)
                def _w():
                    wait_write(j - 2, b)
                step(j, b, (j + 2, b))

        j_t = NF - 4
        wait_write(j_t - 2, 0)
        step(j_t, 0, (j_t + 2, 0))
        wait_write(j_t - 1, 1)
        step(j_t + 1, 1, (j_t + 3, 1))
        wait_write(j_t, 0)
        step(j_t + 2, 0, (NF, 0, TAIL))
        wait_write(j_t + 1, 1)
        step(j_t + 3, 1, None)
        wait_write(j_t + 2, 0)
        step(NF, 0, None, TAIL)
        wait_write(j_t + 3, 1)
        wait_write(NF, 0, TAIL)

    return k(P, Q, rows3, cols3)

    return k(P, Q, rows3d, cols3d)


def _tc_mlp(T, ef, W1e, b1, W2, b2):
    blk = 2560
    F = ef.shape[1]

    def body(t_ref, e_ref, we_ref, b1_ref, w2_ref, b2_ref, o_ref):
        pre = t_ref[...] + jnp.dot(e_ref[...], we_ref[...],
                                   preferred_element_type=jnp.float32)
        hid = jnp.maximum(pre + b1_ref[...], 0.0)
        o_ref[...] = jnp.dot(hid, w2_ref[...],
                             preferred_element_type=jnp.float32) + b2_ref[...]

    return pl.pallas_call(
        body,
        grid=(N_EDGES // blk,),
        in_specs=[
            pl.BlockSpec((blk, D), lambda i: (i, 0)),
            pl.BlockSpec((blk, F), lambda i: (i, 0)),
            pl.BlockSpec((F, D), lambda i: (0, 0)),
            pl.BlockSpec((1, D), lambda i: (0, 0)),
            pl.BlockSpec((D, D), lambda i: (0, 0)),
            pl.BlockSpec((1, D), lambda i: (0, 0)),
        ],
        out_specs=pl.BlockSpec((blk, D), lambda i: (i, 0)),
        out_shape=jax.ShapeDtypeStruct((N_EDGES, D), jnp.float32),
    )(T, ef, W1e, b1, W2, b2)


def _sc_scatter_add(M, rows3d):
    mesh = plsc.VectorSubcoreMesh(core_axis_name="c", subcore_axis_name="s")

    @functools.partial(
        pl.kernel,
        out_type=jax.ShapeDtypeStruct((NC * N_NODES, D), jnp.float32),
        mesh=mesh,
        scratch_types=[
            pltpu.VMEM((CPW, C), jnp.int32),
            pltpu.VMEM((2, C, D), jnp.float32),
            pltpu.VMEM((ZR, D), jnp.float32),
            pltpu.VMEM_SHARED((N_NODES, D), jnp.float32),
            pltpu.SemaphoreType.DMA,
            pltpu.SemaphoreType.DMA,
        ],
    )
    def k(m_hbm, r_hbm, out_hbm, idxv, msgv, zbuf, hacc, rsem0, rsem1):
        cid = lax.axis_index("c")
        sid = lax.axis_index("s")
        wid = sid * NC + cid

        @pl.loop(0, ZR)
        def _z(r):
            for v in range(D // 16):
                zbuf[r, pl.ds(v * 16, 16)] = jnp.zeros((16,), jnp.float32)

        r_base = sid * RPT
        n_cp = jnp.where(sid == NS - 1, (N_NODES - (NS - 1) * RPT) // ZR, RPT // ZR)

        @pl.loop(0, n_cp)
        def _zc(kk):
            pltpu.sync_copy(zbuf, hacc.at[pl.ds(r_base + kk * ZR, ZR)])

        plsc.subcore_barrier()

        pltpu.sync_copy(r_hbm.at[wid], idxv)
        e0 = wid * CPW * C
        rsem = (rsem0, rsem1)

        def issue_read(j, b):
            pltpu.async_copy(m_hbm.at[pl.ds(e0 + j * C, C)], msgv.at[b], rsem[b])

        def wait_read(j, b):
            pltpu.make_async_copy(m_hbm.at[pl.ds(e0 + j * C, C)], msgv.at[b],
                                  rsem[b]).wait()

        issue_read(0, 0)

        @pl.loop(0, CPW - 1, step=2)
        def _chunk(j0):
            for b in (0, 1):
                j = j0 + b
                issue_read(j + 1, 1 - b)
                wait_read(j, b)
                pltpu.sync_copy(msgv.at[b], hacc.at[idxv.at[j]], add=True)

        j_last = CPW - 1
        wait_read(j_last, 0)
        pltpu.sync_copy(msgv.at[0], hacc.at[idxv.at[j_last]], add=True)

        plsc.subcore_barrier()

        @pl.loop(0, n_cp)
        def _wb(kk):
            r0 = r_base + kk * ZR
            pltpu.sync_copy(hacc.at[pl.ds(r0, ZR)],
                            out_hbm.at[pl.ds(cid * N_NODES + r0, ZR)])

    return k(M, rows3d)


def _tc_combine(Hp, delta_row):
    blk = 2000

    def body(a_ref, b_ref, d_ref, o_ref):
        o_ref[...] = a_ref[...] + b_ref[...] + d_ref[...]

    nblk = N_NODES // blk
    return pl.pallas_call(
        body,
        grid=(nblk,),
        in_specs=[
            pl.BlockSpec((blk, D), lambda i: (i, 0)),
            pl.BlockSpec((blk, D), lambda i, n=nblk: (i + n, 0)),
            pl.BlockSpec((1, D), lambda i: (0, 0)),
        ],
        out_specs=pl.BlockSpec((blk, D), lambda i: (i, 0)),
        out_shape=jax.ShapeDtypeStruct((N_NODES, D), jnp.float32),
    )(Hp, Hp, delta_row)


def kernel(h, edge_index, edge_features, n, W1, b1, W2, b2):
    rows = edge_index[0].astype(jnp.int32)
    cols = edge_index[1].astype(jnp.int32)
    rows3d = rows.reshape(NW, CPW, C)
    rows3 = rows.reshape(NW, 1, N_EDGES // NW)
    cols3 = cols.reshape(NW, 1, N_EDGES // NW)
    W1a = W1[:D]
    W1b = W1[D:2 * D]
    W1e = W1[2 * D:]
    b1r = b1.reshape(1, D)
    b2r = b2.reshape(1, D)

    P, Q = _tc_node_transform(h, W1a, W1b)
    T = _sc_gather_add(P, Q, rows3, cols3)
    M = _tc_mlp(T, edge_features, W1e, b1r, W2, b2r)
    Hp = _sc_scatter_add(M, rows3d)
    delta = (jnp.asarray(n) - N_NODES).astype(jnp.float32)
    delta_row = jnp.full((1, D), delta, dtype=jnp.float32)
    return _tc_combine(Hp, delta_row)
